# Initial kernel scaffold; baseline (speedup 1.0000x reference)
import jax
import jax.numpy as jnp
from jax.experimental import pallas as pl


def kernel(qk, v, rotations):
    raise NotImplementedError("write your pallas kernel here")


# speedup vs baseline: 4.1312x; 4.1312x over previous
import functools

import jax
import jax.numpy as jnp
from jax import lax
from jax.experimental import pallas as pl
from jax.experimental.pallas import tpu as pltpu
from jax.experimental.pallas import tpu_sc as plsc

B = 8
S = 4096
D = 64
H = 8
NBK = 64
CS = 64
NC = H * S // CS
CPG = 64
NGRP = NC // CPG
NTOK = B * H * S
SCALE = D ** -0.5

def _hash_dest_kernel(qk_ref, v_ref, rot_ref, gdest_ref, bmask_ref, qkv_ref):
    b = pl.program_id(0)
    x = qk_ref[0]
    rot = rot_ref[...]
    qkv_ref[0] = jnp.concatenate([x, v_ref[0]], axis=1)
    rotated = jnp.dot(x, rot, preferred_element_type=jnp.float32)

    nb = S // CS
    io_r = lax.broadcasted_iota(jnp.int32, (NBK, NBK), 0)
    io_c = lax.broadcasted_iota(jnp.int32, (NBK, NBK), 1)
    upper = (io_r < io_c).astype(jnp.float32)
    lower = (io_c < io_r).astype(jnp.float32)
    iota_v = lax.broadcasted_iota(jnp.int32, (nb, CS, NBK), 2)

    dests = []
    for h in range(H):
        rh = rotated[:, h * 32:(h + 1) * 32]
        full = jnp.concatenate([rh, -rh], axis=1)
        full3 = full.reshape(nb, CS, NBK)
        mx = jnp.max(full3, axis=2, keepdims=True)
        bucket3 = jnp.min(jnp.where(full3 == mx, iota_v, NBK),
                          axis=2, keepdims=True)

        oh3 = (bucket3 == iota_v).astype(jnp.float32)
        cnt = jnp.sum(oh3, axis=1)
        hist = jnp.sum(cnt, axis=0, keepdims=True)
        start = jnp.dot(hist, upper, preferred_element_type=jnp.float32)
        cnt_before = jnp.dot(lower, cnt, preferred_element_type=jnp.float32)

        b_row = jnp.swapaxes(bucket3, 1, 2)
        io_j = lax.broadcasted_iota(jnp.int32, (nb, CS, CS), 1)
        io_k = lax.broadcasted_iota(jnp.int32, (nb, CS, CS), 2)
        cmp = jnp.logical_and(bucket3 == b_row, io_k < io_j)
        rank = jnp.sum(cmp.astype(jnp.float32), axis=2, keepdims=True)

        start_sel = jnp.sum(start.reshape(1, 1, NBK) * oh3, axis=2,
                            keepdims=True)
        cntb_sel = jnp.sum(cnt_before[:, None, :] * oh3, axis=2,
                           keepdims=True)
        dest = (start_sel + cntb_sel + rank).astype(jnp.int32)
        dests.append(dest)
        gdest_ref[0, h] = dest[:, :, 0] + (b * H + h) * S

    for h in range(H):
        dcur = dests[h]
        dprev = dests[(h - 1) % H]
        a = (dcur == iota_v).astype(jnp.float32).reshape(S, NBK)
        bb = (dprev == iota_v + (S - CS)).astype(jnp.float32).reshape(S, NBK)
        m = lax.dot_general(a, bb, (((0,), (0,)), ((), ())),
                            preferred_element_type=jnp.float32)
        bmask_ref[0, h] = m


def _hash_dest(qk, v, rot2):
    return pl.pallas_call(
        _hash_dest_kernel,
        grid=(B,),
        in_specs=[
            pl.BlockSpec((1, S, D), lambda b: (b, 0, 0)),
            pl.BlockSpec((1, S, D), lambda b: (b, 0, 0)),
            pl.BlockSpec((D, H * 32), lambda b: (0, 0)),
        ],
        out_specs=[
            pl.BlockSpec((1, H, S // CS, CS), lambda b: (b, 0, 0, 0)),
            pl.BlockSpec((1, H, CS, CS), lambda b: (b, 0, 0, 0)),
            pl.BlockSpec((1, S, 2 * D), lambda b: (b, 0, 0)),
        ],
        out_shape=[
            jax.ShapeDtypeStruct((B, H, S // CS, CS), jnp.int32),
            jax.ShapeDtypeStruct((B, H, CS, CS), jnp.float32),
            jax.ShapeDtypeStruct((B, S, 2 * D), jnp.float32),
        ],
    )(qk, v, rot2)


_NW = 32
_W = 256


def _sc_mesh():
    return plsc.VectorSubcoreMesh(core_axis_name="c", subcore_axis_name="s")


def _sc_scatter(qkv, gidx):
    @functools.partial(
        pl.kernel,
        mesh=_sc_mesh(),
        out_type=jax.ShapeDtypeStruct((NTOK, 2 * D), jnp.float32),
        scratch_types=[
            pltpu.VMEM((_W, 2 * D), jnp.float32),
            pltpu.VMEM((_W,), jnp.int32),
        ],
    )
    def k(qkv_hbm, gidx_hbm, sqkv_hbm, rows_v, idx_v):
        wid = lax.axis_index("s") * 2 + lax.axis_index("c")
        b = wid // 4
        quarter = wid % 4

        @pl.loop(0, 1024 // _W)
        def _(ci):
            t0 = quarter * 1024 + ci * _W
            pltpu.sync_copy(qkv_hbm.at[pl.ds(b * S + t0, _W)], rows_v)
            for h in range(H):
                pltpu.sync_copy(gidx_hbm.at[pl.ds((b * H + h) * S + t0, _W)],
                                idx_v)
                pltpu.sync_copy(rows_v, sqkv_hbm.at[idx_v])

    return k(qkv, gidx)


def _sc_gather(so_ext, gidx):
    @functools.partial(
        pl.kernel,
        mesh=_sc_mesh(),
        out_type=jax.ShapeDtypeStruct((NTOK, 2 * D), jnp.float32),
        scratch_types=[
            pltpu.VMEM((_W, 2 * D), jnp.float32),
            pltpu.VMEM((_W,), jnp.int32),
        ],
    )
    def k(so_hbm, gidx_hbm, oext_hbm, rows_v, idx_v):
        wid = lax.axis_index("s") * 2 + lax.axis_index("c")
        per_w = NTOK // _NW

        @pl.loop(0, per_w // _W)
        def _(ci):
            g0 = wid * per_w + ci * _W
            pltpu.sync_copy(gidx_hbm.at[pl.ds(g0, _W)], idx_v)
            pltpu.sync_copy(so_hbm.at[idx_v], rows_v)
            pltpu.sync_copy(rows_v, oext_hbm.at[pl.ds(g0, _W)])

    return k(so_ext, gidx)


def _attn_kernel(sqkv_ref, halo_ref, bmask_ref, out_ref):
    ii = lax.broadcasted_iota(jnp.int32, (CS, CS), 0)
    jj = lax.broadcasted_iota(jnp.int32, (CS, CS), 1)
    eye = ii == jj

    def nrm(x):
        n = jnp.sqrt(jnp.sum(x * x, axis=1, keepdims=True))
        return x / jnp.maximum(n, 1e-12)

    def chunk(cur, prev, pmask):
        q = cur[:, :D]
        vcur = cur[:, D:]
        kcur = nrm(q)
        kprev = nrm(prev[:, :D])
        vprev = prev[:, D:]

        dots_c = lax.dot_general(q, kcur, (((1,), (1,)), ((), ())),
                                 preferred_element_type=jnp.float32) * SCALE
        dots_p = lax.dot_general(q, kprev, (((1,), (1,)), ((), ())),
                                 preferred_element_type=jnp.float32) * SCALE
        dots_c = jnp.where(eye, -50000.0, dots_c)
        if pmask is not None:
            dots_p = jnp.where(pmask > 0.5, -50000.0, dots_p)

        m = jnp.maximum(jnp.max(dots_c, axis=1, keepdims=True),
                        jnp.max(dots_p, axis=1, keepdims=True))
        pc = jnp.exp(dots_c - m)
        pp = jnp.exp(dots_p - m)
        ssum = (jnp.sum(pc, axis=1, keepdims=True)
                + jnp.sum(pp, axis=1, keepdims=True))
        lse = m + jnp.log(ssum)
        o = (jnp.dot(pc, vcur, preferred_element_type=jnp.float32)
             + jnp.dot(pp, vprev, preferred_element_type=jnp.float32)) / ssum
        return jnp.concatenate([o, jnp.broadcast_to(lse, (CS, D))], axis=1)

    out_ref[0, 0] = chunk(sqkv_ref[0, 0], halo_ref[0, 0], bmask_ref[0, 0])

    def body(c, _):
        out_ref[0, c] = chunk(sqkv_ref[0, c], sqkv_ref[0, c - 1], None)
        return 0

    lax.fori_loop(1, CPG, body, 0)


def _attention(sqkv4, bmask):
    return pl.pallas_call(
        _attn_kernel,
        grid=(B, NGRP),
        in_specs=[
            pl.BlockSpec((1, CPG, CS, 2 * D), lambda b, g: (b, g, 0, 0)),
            pl.BlockSpec((1, 1, CS, 2 * D),
                         lambda b, g: (b, (g * CPG - 1) % NC, 0, 0)),
            pl.BlockSpec((1, 1, CS, CS), lambda b, g: (b, g, 0, 0)),
        ],
        out_specs=pl.BlockSpec((1, CPG, CS, 2 * D), lambda b, g: (b, g, 0, 0)),
        out_shape=jax.ShapeDtypeStruct((B, NC, CS, 2 * D), jnp.float32),
    )(sqkv4, sqkv4, bmask)


_T = 512


def _combine_kernel(oext_ref, out_ref):
    x = oext_ref[0]
    o = x[:, :, :D]
    l = x[:, :, D:D + 1]
    m = jnp.max(l, axis=0, keepdims=True)
    w = jnp.exp(l - m)
    s = jnp.sum(w, axis=0)
    acc = jnp.sum(o * w, axis=0)
    out_ref[0] = acc / s


def _combine(o_ext4):
    return pl.pallas_call(
        _combine_kernel,
        grid=(B, S // _T),
        in_specs=[pl.BlockSpec((1, H, _T, 2 * D), lambda b, t: (b, 0, t, 0))],
        out_specs=pl.BlockSpec((1, _T, D), lambda b, t: (b, t, 0)),
        out_shape=jax.ShapeDtypeStruct((B, S, D), jnp.float32),
    )(o_ext4)


def kernel(qk, v, rotations):
    rot2 = rotations.reshape(D, H * 32)
    gdest4, bmask, qkv3 = _hash_dest(qk, v, rot2)
    gdest = gdest4.reshape(NTOK)

    sqkv = _sc_scatter(qkv3.reshape(B * S, 2 * D), gdest)

    sqkv4 = sqkv.reshape(B, NC, CS, 2 * D)
    so_ext = _attention(sqkv4, bmask)

    o_ext = _sc_gather(so_ext.reshape(NTOK, 2 * D), gdest)
    out = _combine(o_ext.reshape(B, H, S, 2 * D))
    return out


# speedup vs baseline: 5.5800x; 1.3507x over previous
import functools

import jax
import jax.numpy as jnp
from jax import lax
from jax.experimental import pallas as pl
from jax.experimental.pallas import tpu as pltpu
from jax.experimental.pallas import tpu_sc as plsc

B = 8
S = 4096
D = 64
H = 8
NBK = 64
CS = 64
NC = H * S // CS
CPG = 64
NGRP = NC // CPG
NTOK = B * H * S
SCALE = D ** -0.5

def _hash_dest_kernel(qk_ref, v_ref, rot_ref, gdest_ref, bmask_ref, qkv_ref):
    b = pl.program_id(0)
    x = qk_ref[0]
    rot = rot_ref[...]
    qkv_ref[0] = jnp.concatenate([x, v_ref[0]], axis=1)
    rotated = jnp.dot(x, rot, preferred_element_type=jnp.float32)

    nb = S // CS
    io_r = lax.broadcasted_iota(jnp.int32, (NBK, NBK), 0)
    io_c = lax.broadcasted_iota(jnp.int32, (NBK, NBK), 1)
    upper = (io_r < io_c).astype(jnp.float32)
    lower = (io_c < io_r).astype(jnp.float32)
    iota_v = lax.broadcasted_iota(jnp.int32, (nb, CS, NBK), 2)

    dests = []
    for h in range(H):
        rh = rotated[:, h * 32:(h + 1) * 32]
        full = jnp.concatenate([rh, -rh], axis=1)
        full3 = full.reshape(nb, CS, NBK)
        mx = jnp.max(full3, axis=2, keepdims=True)
        bucket3 = jnp.min(jnp.where(full3 == mx, iota_v, NBK),
                          axis=2, keepdims=True)

        oh3 = (bucket3 == iota_v).astype(jnp.float32)
        cnt = jnp.sum(oh3, axis=1)
        hist = jnp.sum(cnt, axis=0, keepdims=True)
        start = jnp.dot(hist, upper, preferred_element_type=jnp.float32)
        cnt_before = jnp.dot(lower, cnt, preferred_element_type=jnp.float32)

        b_row = jnp.swapaxes(bucket3, 1, 2)
        io_j = lax.broadcasted_iota(jnp.int32, (nb, CS, CS), 1)
        io_k = lax.broadcasted_iota(jnp.int32, (nb, CS, CS), 2)
        cmp = jnp.logical_and(bucket3 == b_row, io_k < io_j)
        rank = jnp.sum(cmp.astype(jnp.float32), axis=2, keepdims=True)

        start_sel = jnp.sum(start.reshape(1, 1, NBK) * oh3, axis=2,
                            keepdims=True)
        cntb_sel = jnp.sum(cnt_before[:, None, :] * oh3, axis=2,
                           keepdims=True)
        dest = (start_sel + cntb_sel + rank).astype(jnp.int32)
        dests.append(dest)
        gdest_ref[0, h] = dest[:, :, 0] + (b * H + h) * S

    for h in range(H):
        dcur = dests[h]
        dprev = dests[(h - 1) % H]
        a = (dcur == iota_v).astype(jnp.float32).reshape(S, NBK)
        bb = (dprev == iota_v + (S - CS)).astype(jnp.float32).reshape(S, NBK)
        m = lax.dot_general(a, bb, (((0,), (0,)), ((), ())),
                            preferred_element_type=jnp.float32)
        bmask_ref[0, h] = m


def _hash_dest(qk, v, rot2):
    return pl.pallas_call(
        _hash_dest_kernel,
        grid=(B,),
        in_specs=[
            pl.BlockSpec((1, S, D), lambda b: (b, 0, 0)),
            pl.BlockSpec((1, S, D), lambda b: (b, 0, 0)),
            pl.BlockSpec((D, H * 32), lambda b: (0, 0)),
        ],
        out_specs=[
            pl.BlockSpec((1, H, S // CS, CS), lambda b: (b, 0, 0, 0)),
            pl.BlockSpec((1, H, CS, CS), lambda b: (b, 0, 0, 0)),
            pl.BlockSpec((1, S, 2 * D), lambda b: (b, 0, 0)),
        ],
        out_shape=[
            jax.ShapeDtypeStruct((B, H, S // CS, CS), jnp.int32),
            jax.ShapeDtypeStruct((B, H, CS, CS), jnp.float32),
            jax.ShapeDtypeStruct((B, S, 2 * D), jnp.float32),
        ],
    )(qk, v, rot2)


_NW = 32
_W = 256


def _sc_mesh():
    return plsc.VectorSubcoreMesh(core_axis_name="c", subcore_axis_name="s")


def _sc_scatter(qkv, gidx):
    @functools.partial(
        pl.kernel,
        mesh=_sc_mesh(),
        out_type=jax.ShapeDtypeStruct((NTOK, 2 * D), jnp.float32),
        scratch_types=[
            pltpu.VMEM((_W, 2 * D), jnp.float32),
            pltpu.VMEM((_W,), jnp.int32),
        ],
    )
    def k(qkv_hbm, gidx_hbm, sqkv_hbm, rows_v, idx_v):
        wid = lax.axis_index("s") * 2 + lax.axis_index("c")
        b = wid // 4
        quarter = wid % 4

        @pl.loop(0, 1024 // _W)
        def _(ci):
            t0 = quarter * 1024 + ci * _W
            pltpu.sync_copy(qkv_hbm.at[pl.ds(b * S + t0, _W)], rows_v)
            for h in range(H):
                pltpu.sync_copy(gidx_hbm.at[pl.ds((b * H + h) * S + t0, _W)],
                                idx_v)
                pltpu.sync_copy(rows_v, sqkv_hbm.at[idx_v])

    return k(qkv, gidx)


def _sc_gather(so_ext, gidx):
    @functools.partial(
        pl.kernel,
        mesh=_sc_mesh(),
        out_type=jax.ShapeDtypeStruct((NTOK, 2 * D), jnp.float32),
        scratch_types=[
            pltpu.VMEM((_W, 2 * D), jnp.float32),
            pltpu.VMEM((_W,), jnp.int32),
        ],
    )
    def k(so_hbm, gidx_hbm, oext_hbm, rows_v, idx_v):
        wid = lax.axis_index("s") * 2 + lax.axis_index("c")
        per_w = NTOK // _NW

        @pl.loop(0, per_w // _W)
        def _(ci):
            g0 = wid * per_w + ci * _W
            pltpu.sync_copy(gidx_hbm.at[pl.ds(g0, _W)], idx_v)
            pltpu.sync_copy(so_hbm.at[idx_v], rows_v)
            pltpu.sync_copy(rows_v, oext_hbm.at[pl.ds(g0, _W)])

    return k(so_ext, gidx)


def _attn_kernel(sqkv_ref, halo_ref, bmask_ref, out_ref, q_ref, k_ref, v_ref):
    ii = lax.broadcasted_iota(jnp.int32, (CS, CS), 0)
    jj = lax.broadcasted_iota(jnp.int32, (CS, CS), 1)
    eye = ii == jj

    def nrm(x):
        n = jnp.sqrt(jnp.sum(x * x, axis=1, keepdims=True))
        return x / jnp.maximum(n, 1e-12)

    x2 = sqkv_ref[0].reshape(CPG * CS, 2 * D)
    halo = halo_ref[0, 0]
    q_all = x2[:, :D]
    q_ref[...] = q_all.astype(jnp.bfloat16)
    k_ref[...] = nrm(q_all).astype(jnp.bfloat16)
    v_ref[...] = x2[:, D:].astype(jnp.bfloat16)

    def chunk(q, kcur, vcur, kprev, vprev, pmask):
        dots_c = lax.dot_general(q, kcur, (((1,), (1,)), ((), ())),
                                 preferred_element_type=jnp.float32) * SCALE
        dots_p = lax.dot_general(q, kprev, (((1,), (1,)), ((), ())),
                                 preferred_element_type=jnp.float32) * SCALE
        dots_c = jnp.where(eye, -50000.0, dots_c)
        if pmask is not None:
            dots_p = jnp.where(pmask > 0.5, -50000.0, dots_p)

        m = jnp.maximum(jnp.max(dots_c, axis=1, keepdims=True),
                        jnp.max(dots_p, axis=1, keepdims=True))
        pc = jnp.exp(dots_c - m)
        pp = jnp.exp(dots_p - m)
        ssum = (jnp.sum(pc, axis=1, keepdims=True)
                + jnp.sum(pp, axis=1, keepdims=True))
        lse = m + jnp.log(ssum)
        o = (jnp.dot(pc.astype(jnp.bfloat16), vcur,
                     preferred_element_type=jnp.float32)
             + jnp.dot(pp.astype(jnp.bfloat16), vprev,
                       preferred_element_type=jnp.float32)) / ssum
        return jnp.concatenate([o, jnp.broadcast_to(lse, (CS, D))], axis=1)

    def piece(c):
        sl = pl.ds(c * CS, CS)
        return q_ref[sl, :], k_ref[sl, :], v_ref[sl, :]

    q0, k0, v0 = piece(0)
    out_ref[0, 0] = chunk(q0, k0, v0,
                          nrm(halo[:, :D]).astype(jnp.bfloat16),
                          halo[:, D:].astype(jnp.bfloat16),
                          bmask_ref[0, 0])

    UNROLL = 4

    def body(i, _):
        c0 = 1 + i * UNROLL
        for u in range(UNROLL):
            c = c0 + u
            qc, kc, vc = piece(c)
            kp, vp = k_ref[pl.ds((c - 1) * CS, CS), :], \
                v_ref[pl.ds((c - 1) * CS, CS), :]
            out_ref[0, c] = chunk(qc, kc, vc, kp, vp, None)
        return 0

    lax.fori_loop(0, (CPG - 4) // UNROLL, body, 0)
    for c in range(CPG - 3, CPG):
        qc, kc, vc = piece(c)
        kp, vp = piece(c - 1)[1], piece(c - 1)[2]
        out_ref[0, c] = chunk(qc, kc, vc, kp, vp, None)


def _attention(sqkv4, bmask):
    return pl.pallas_call(
        _attn_kernel,
        grid=(B, NGRP),
        in_specs=[
            pl.BlockSpec((1, CPG, CS, 2 * D), lambda b, g: (b, g, 0, 0)),
            pl.BlockSpec((1, 1, CS, 2 * D),
                         lambda b, g: (b, (g * CPG - 1) % NC, 0, 0)),
            pl.BlockSpec((1, 1, CS, CS), lambda b, g: (b, g, 0, 0)),
        ],
        out_specs=pl.BlockSpec((1, CPG, CS, 2 * D), lambda b, g: (b, g, 0, 0)),
        out_shape=jax.ShapeDtypeStruct((B, NC, CS, 2 * D), jnp.float32),
        scratch_shapes=[
            pltpu.VMEM((CPG * CS, D), jnp.bfloat16),
            pltpu.VMEM((CPG * CS, D), jnp.bfloat16),
            pltpu.VMEM((CPG * CS, D), jnp.bfloat16),
        ],
    )(sqkv4, sqkv4, bmask)


_T = 512


def _combine_kernel(oext_ref, out_ref):
    x = oext_ref[0]
    o = x[:, :, :D]
    l = x[:, :, D:D + 1]
    m = jnp.max(l, axis=0, keepdims=True)
    w = jnp.exp(l - m)
    s = jnp.sum(w, axis=0)
    acc = jnp.sum(o * w, axis=0)
    out_ref[0] = acc / s


def _combine(o_ext4):
    return pl.pallas_call(
        _combine_kernel,
        grid=(B, S // _T),
        in_specs=[pl.BlockSpec((1, H, _T, 2 * D), lambda b, t: (b, 0, t, 0))],
        out_specs=pl.BlockSpec((1, _T, D), lambda b, t: (b, t, 0)),
        out_shape=jax.ShapeDtypeStruct((B, S, D), jnp.float32),
    )(o_ext4)


def kernel(qk, v, rotations):
    rot2 = rotations.reshape(D, H * 32)
    gdest4, bmask, qkv3 = _hash_dest(qk, v, rot2)
    gdest = gdest4.reshape(NTOK)

    sqkv = _sc_scatter(qkv3.reshape(B * S, 2 * D), gdest)

    sqkv4 = sqkv.reshape(B, NC, CS, 2 * D)
    so_ext = _attention(sqkv4, bmask)

    o_ext = _sc_gather(so_ext.reshape(NTOK, 2 * D), gdest)
    out = _combine(o_ext.reshape(B, H, S, 2 * D))
    return out


# speedup vs baseline: 10.9435x; 1.9612x over previous
import functools

import jax
import jax.numpy as jnp
from jax import lax
from jax.experimental import pallas as pl
from jax.experimental.pallas import tpu as pltpu
from jax.experimental.pallas import tpu_sc as plsc

B = 8
S = 4096
D = 64
H = 8
NBK = 64
CS = 64
NC = H * S // CS
CPG = 64
NGRP = NC // CPG
NTOK = B * H * S
SCALE = D ** -0.5

def _hash_dest_kernel(qk_ref, v_ref, rot_ref, gdest_ref, bmask_ref, qkv_ref):
    b = pl.program_id(0)
    x = qk_ref[0]
    rot = rot_ref[...]
    qkv_ref[0] = jnp.concatenate([x, v_ref[0]], axis=1)
    rotated = jnp.dot(x, rot, preferred_element_type=jnp.float32)

    nb = S // CS
    io_r = lax.broadcasted_iota(jnp.int32, (NBK, NBK), 0)
    io_c = lax.broadcasted_iota(jnp.int32, (NBK, NBK), 1)
    upper = (io_r < io_c).astype(jnp.float32)
    lower = (io_c < io_r).astype(jnp.float32)
    iota_v = lax.broadcasted_iota(jnp.int32, (nb, CS, NBK), 2)

    dests = []
    for h in range(H):
        rh = rotated[:, h * 32:(h + 1) * 32]
        full = jnp.concatenate([rh, -rh], axis=1)
        full3 = full.reshape(nb, CS, NBK)
        mx = jnp.max(full3, axis=2, keepdims=True)
        bucket3 = jnp.min(jnp.where(full3 == mx, iota_v, NBK),
                          axis=2, keepdims=True)

        oh3 = (bucket3 == iota_v).astype(jnp.float32)
        cnt = jnp.sum(oh3, axis=1)
        hist = jnp.sum(cnt, axis=0, keepdims=True)
        start = jnp.dot(hist, upper, preferred_element_type=jnp.float32)
        cnt_before = jnp.dot(lower, cnt, preferred_element_type=jnp.float32)

        b_row = jnp.swapaxes(bucket3, 1, 2)
        io_j = lax.broadcasted_iota(jnp.int32, (nb, CS, CS), 1)
        io_k = lax.broadcasted_iota(jnp.int32, (nb, CS, CS), 2)
        cmp = jnp.logical_and(bucket3 == b_row, io_k < io_j)
        rank = jnp.sum(cmp.astype(jnp.float32), axis=2, keepdims=True)

        start_sel = jnp.sum(start.reshape(1, 1, NBK) * oh3, axis=2,
                            keepdims=True)
        cntb_sel = jnp.sum(cnt_before[:, None, :] * oh3, axis=2,
                           keepdims=True)
        dest = (start_sel + cntb_sel + rank).astype(jnp.int32)
        dests.append(dest)
        gdest_ref[0, h] = dest[:, :, 0] + (b * H + h) * S

    for h in range(H):
        dcur = dests[h]
        dprev = dests[(h - 1) % H]
        a = (dcur == iota_v).astype(jnp.float32).reshape(S, NBK)
        bb = (dprev == iota_v + (S - CS)).astype(jnp.float32).reshape(S, NBK)
        m = lax.dot_general(a, bb, (((0,), (0,)), ((), ())),
                            preferred_element_type=jnp.float32)
        bmask_ref[0, h] = m


def _hash_dest(qk, v, rot2):
    return pl.pallas_call(
        _hash_dest_kernel,
        grid=(B,),
        in_specs=[
            pl.BlockSpec((1, S, D), lambda b: (b, 0, 0)),
            pl.BlockSpec((1, S, D), lambda b: (b, 0, 0)),
            pl.BlockSpec((D, H * 32), lambda b: (0, 0)),
        ],
        out_specs=[
            pl.BlockSpec((1, H, S // CS, CS), lambda b: (b, 0, 0, 0)),
            pl.BlockSpec((1, H, CS, CS), lambda b: (b, 0, 0, 0)),
            pl.BlockSpec((1, S, 2 * D), lambda b: (b, 0, 0)),
        ],
        out_shape=[
            jax.ShapeDtypeStruct((B, H, S // CS, CS), jnp.int32),
            jax.ShapeDtypeStruct((B, H, CS, CS), jnp.float32),
            jax.ShapeDtypeStruct((B, S, 2 * D), jnp.float32),
        ],
    )(qk, v, rot2)


_NW = 32
_W = 256


def _sc_mesh():
    return plsc.VectorSubcoreMesh(core_axis_name="c", subcore_axis_name="s")


def _sc_scatter(qkv, gidx):
    @functools.partial(
        pl.kernel,
        mesh=_sc_mesh(),
        out_type=jax.ShapeDtypeStruct((NTOK, 2 * D), jnp.float32),
        scratch_types=[
            pltpu.VMEM((_W, 2 * D), jnp.float32),
            pltpu.VMEM((_W,), jnp.int32),
        ],
    )
    def k(qkv_hbm, gidx_hbm, sqkv_hbm, rows_v, idx_v):
        wid = lax.axis_index("s") * 2 + lax.axis_index("c")
        b = wid // 4
        quarter = wid % 4

        @pl.loop(0, 1024 // _W)
        def _(ci):
            t0 = quarter * 1024 + ci * _W
            pltpu.sync_copy(qkv_hbm.at[pl.ds(b * S + t0, _W)], rows_v)
            for h in range(H):
                pltpu.sync_copy(gidx_hbm.at[pl.ds((b * H + h) * S + t0, _W)],
                                idx_v)
                pltpu.sync_copy(rows_v, sqkv_hbm.at[idx_v])

    return k(qkv, gidx)


def _sc_gather(so_ext, gidx):
    @functools.partial(
        pl.kernel,
        mesh=_sc_mesh(),
        out_type=jax.ShapeDtypeStruct((NTOK, 2 * D), jnp.float32),
        scratch_types=[
            pltpu.VMEM((_W, 2 * D), jnp.float32),
            pltpu.VMEM((_W,), jnp.int32),
        ],
    )
    def k(so_hbm, gidx_hbm, oext_hbm, rows_v, idx_v):
        wid = lax.axis_index("s") * 2 + lax.axis_index("c")
        per_w = NTOK // _NW

        @pl.loop(0, per_w // _W)
        def _(ci):
            g0 = wid * per_w + ci * _W
            pltpu.sync_copy(gidx_hbm.at[pl.ds(g0, _W)], idx_v)
            pltpu.sync_copy(so_hbm.at[idx_v], rows_v)
            pltpu.sync_copy(rows_v, oext_hbm.at[pl.ds(g0, _W)])

    return k(so_ext, gidx)


def _attn_kernel(sqkv_ref, halo_ref, bmask_ref, out_ref,
                 q_ref, k_ref, v_ref, d_ref, p_ref):
    def nrm(x):
        n = jnp.sqrt(jnp.sum(x * x, axis=1, keepdims=True))
        return x / jnp.maximum(n, 1e-12)

    x2 = sqkv_ref[0].reshape(CPG * CS, 2 * D)
    halo = halo_ref[0, 0]
    q_all = x2[:, :D]
    q_ref[...] = q_all.astype(jnp.bfloat16)
    k_ref[0:CS, :] = nrm(halo[:, :D]).astype(jnp.bfloat16)
    k_ref[CS:, :] = nrm(q_all).astype(jnp.bfloat16)
    v_ref[0:CS, :] = halo[:, D:].astype(jnp.bfloat16)
    v_ref[CS:, :] = x2[:, D:].astype(jnp.bfloat16)

    for c in range(CPG):
        d_ref[c * CS:(c + 1) * CS, :] = lax.dot_general(
            q_ref[c * CS:(c + 1) * CS, :], k_ref[c * CS:(c + 2) * CS, :],
            (((1,), (1,)), ((), ())), preferred_element_type=jnp.float32)

    dots = d_ref[...] * SCALE
    ii = lax.broadcasted_iota(jnp.int32, (CPG * CS, 2 * CS), 0)
    jj = lax.broadcasted_iota(jnp.int32, (CPG * CS, 2 * CS), 1)
    eye = jj == (ii % CS) + CS
    dots = jnp.where(eye, -50000.0, dots)
    bpad = jnp.pad(bmask_ref[0, 0], ((0, CPG * CS - CS), (0, CS)))
    dots = jnp.where(bpad > 0.5, -50000.0, dots)
    m = jnp.max(dots, axis=1, keepdims=True)
    pexp = jnp.exp(dots - m)
    ssum = jnp.sum(pexp, axis=1, keepdims=True)
    lse = m + jnp.log(ssum)
    p_ref[...] = (pexp / ssum).astype(jnp.bfloat16)
    out_ref[0, :, :, D:] = jnp.broadcast_to(
        lse.reshape(CPG, CS, 1), (CPG, CS, D))

    for c in range(CPG):
        out_ref[0, c, :, :D] = lax.dot_general(
            p_ref[c * CS:(c + 1) * CS, :], v_ref[c * CS:(c + 2) * CS, :],
            (((1,), (0,)), ((), ())), preferred_element_type=jnp.float32)


def _attention(sqkv4, bmask):
    return pl.pallas_call(
        _attn_kernel,
        grid=(B, NGRP),
        in_specs=[
            pl.BlockSpec((1, CPG, CS, 2 * D), lambda b, g: (b, g, 0, 0)),
            pl.BlockSpec((1, 1, CS, 2 * D),
                         lambda b, g: (b, (g * CPG - 1) % NC, 0, 0)),
            pl.BlockSpec((1, 1, CS, CS), lambda b, g: (b, g, 0, 0)),
        ],
        out_specs=pl.BlockSpec((1, CPG, CS, 2 * D), lambda b, g: (b, g, 0, 0)),
        out_shape=jax.ShapeDtypeStruct((B, NC, CS, 2 * D), jnp.float32),
        scratch_shapes=[
            pltpu.VMEM((CPG * CS, D), jnp.bfloat16),
            pltpu.VMEM(((CPG + 1) * CS, D), jnp.bfloat16),
            pltpu.VMEM(((CPG + 1) * CS, D), jnp.bfloat16),
            pltpu.VMEM((CPG * CS, 2 * CS), jnp.float32),
            pltpu.VMEM((CPG * CS, 2 * CS), jnp.bfloat16),
        ],
    )(sqkv4, sqkv4, bmask)


_T = 512


def _combine_kernel(oext_ref, out_ref):
    x = oext_ref[0]
    o = x[:, :, :D]
    l = x[:, :, D:D + 1]
    m = jnp.max(l, axis=0, keepdims=True)
    w = jnp.exp(l - m)
    s = jnp.sum(w, axis=0)
    acc = jnp.sum(o * w, axis=0)
    out_ref[0] = acc / s


def _combine(o_ext4):
    return pl.pallas_call(
        _combine_kernel,
        grid=(B, S // _T),
        in_specs=[pl.BlockSpec((1, H, _T, 2 * D), lambda b, t: (b, 0, t, 0))],
        out_specs=pl.BlockSpec((1, _T, D), lambda b, t: (b, t, 0)),
        out_shape=jax.ShapeDtypeStruct((B, S, D), jnp.float32),
    )(o_ext4)


def kernel(qk, v, rotations):
    rot2 = rotations.reshape(D, H * 32)
    gdest4, bmask, qkv3 = _hash_dest(qk, v, rot2)
    gdest = gdest4.reshape(NTOK)

    sqkv = _sc_scatter(qkv3.reshape(B * S, 2 * D), gdest)

    sqkv4 = sqkv.reshape(B, NC, CS, 2 * D)
    so_ext = _attention(sqkv4, bmask)

    o_ext = _sc_gather(so_ext.reshape(NTOK, 2 * D), gdest)
    out = _combine(o_ext.reshape(B, H, S, 2 * D))
    return out


# speedup vs baseline: 10.9926x; 1.0045x over previous
import functools

import jax
import jax.numpy as jnp
from jax import lax
from jax.experimental import pallas as pl
from jax.experimental.pallas import tpu as pltpu
from jax.experimental.pallas import tpu_sc as plsc

B = 8
S = 4096
D = 64
H = 8
NBK = 64
CS = 64
NC = H * S // CS
CPG = 64
NGRP = NC // CPG
NTOK = B * H * S
SCALE = D ** -0.5

def _hash_dest_kernel(qk_ref, v_ref, rot_ref, gdest_ref, bmask_ref, qkv_ref):
    b = pl.program_id(0)
    x = qk_ref[0]
    rot = rot_ref[...]
    qkv_ref[0] = jnp.concatenate([x, v_ref[0]], axis=1)
    rotated = jnp.dot(x, rot, preferred_element_type=jnp.float32)

    SB = 8
    NSB = S // SB
    io_r = lax.broadcasted_iota(jnp.int32, (NBK, NBK), 0)
    io_c = lax.broadcasted_iota(jnp.int32, (NBK, NBK), 1)
    upper = (io_r < io_c).astype(jnp.float32)
    io_R = lax.broadcasted_iota(jnp.int32, (NSB, NSB), 0)
    io_C = lax.broadcasted_iota(jnp.int32, (NSB, NSB), 1)
    lowerS = (io_C < io_R).astype(jnp.float32)
    iota_v = lax.broadcasted_iota(jnp.int32, (NSB, SB, NBK), 2)
    io_j = lax.broadcasted_iota(jnp.int32, (NSB, SB, SB), 1)
    io_k = lax.broadcasted_iota(jnp.int32, (NSB, SB, SB), 2)
    jlt = io_k < io_j

    dests = []
    for h in range(H):
        rh = rotated[:, h * 32:(h + 1) * 32]
        full = jnp.concatenate([rh, -rh], axis=1)
        full3 = full.reshape(NSB, SB, NBK)
        mx = jnp.max(full3, axis=2, keepdims=True)
        bucket3 = jnp.min(jnp.where(full3 == mx, iota_v, NBK),
                          axis=2, keepdims=True)

        oh3 = (bucket3 == iota_v).astype(jnp.float32)
        cnt = jnp.sum(oh3, axis=1)
        hist = jnp.sum(cnt, axis=0, keepdims=True)
        start = jnp.dot(hist, upper, preferred_element_type=jnp.float32)
        cnt_before = jnp.dot(lowerS, cnt, preferred_element_type=jnp.float32)

        b_row = jnp.swapaxes(bucket3, 1, 2)
        cmp = jnp.logical_and(bucket3 == b_row, jlt)
        rank = jnp.sum(cmp.astype(jnp.float32), axis=2, keepdims=True)

        base = start + cnt_before
        sel = jnp.sum(base[:, None, :] * oh3, axis=2, keepdims=True)
        dest = (sel + rank).astype(jnp.int32)
        dests.append(dest)
        gdest_ref[0, h] = dest[:, :, 0] + (b * H + h) * S

    for h in range(H):
        dcur = dests[h]
        dprev = dests[(h - 1) % H]
        a = (dcur == iota_v).astype(jnp.float32).reshape(S, NBK)
        bb = (dprev == iota_v + (S - CS)).astype(jnp.float32).reshape(S, NBK)
        m = lax.dot_general(a, bb, (((0,), (0,)), ((), ())),
                            preferred_element_type=jnp.float32)
        bmask_ref[0, h] = m


def _hash_dest(qk, v, rot2):
    return pl.pallas_call(
        _hash_dest_kernel,
        grid=(B,),
        in_specs=[
            pl.BlockSpec((1, S, D), lambda b: (b, 0, 0)),
            pl.BlockSpec((1, S, D), lambda b: (b, 0, 0)),
            pl.BlockSpec((D, H * 32), lambda b: (0, 0)),
        ],
        out_specs=[
            pl.BlockSpec((1, H, S // 8, 8), lambda b: (b, 0, 0, 0)),
            pl.BlockSpec((1, H, CS, CS), lambda b: (b, 0, 0, 0)),
            pl.BlockSpec((1, S, 2 * D), lambda b: (b, 0, 0)),
        ],
        out_shape=[
            jax.ShapeDtypeStruct((B, H, S // 8, 8), jnp.int32),
            jax.ShapeDtypeStruct((B, H, CS, CS), jnp.float32),
            jax.ShapeDtypeStruct((B, S, 2 * D), jnp.float32),
        ],
    )(qk, v, rot2)


_NW = 32
_W = 256


def _sc_mesh():
    return plsc.VectorSubcoreMesh(core_axis_name="c", subcore_axis_name="s")


def _sc_scatter(qkv, gidx):
    @functools.partial(
        pl.kernel,
        mesh=_sc_mesh(),
        out_type=jax.ShapeDtypeStruct((NTOK, 2 * D), jnp.float32),
        scratch_types=[
            pltpu.VMEM((_W, 2 * D), jnp.float32),
            pltpu.VMEM((_W,), jnp.int32),
        ],
    )
    def k(qkv_hbm, gidx_hbm, sqkv_hbm, rows_v, idx_v):
        wid = lax.axis_index("s") * 2 + lax.axis_index("c")
        b = wid // 4
        quarter = wid % 4

        @pl.loop(0, 1024 // _W)
        def _(ci):
            t0 = quarter * 1024 + ci * _W
            pltpu.sync_copy(qkv_hbm.at[pl.ds(b * S + t0, _W)], rows_v)
            for h in range(H):
                pltpu.sync_copy(gidx_hbm.at[pl.ds((b * H + h) * S + t0, _W)],
                                idx_v)
                pltpu.sync_copy(rows_v, sqkv_hbm.at[idx_v])

    return k(qkv, gidx)


def _sc_gather(so_ext, gidx):
    @functools.partial(
        pl.kernel,
        mesh=_sc_mesh(),
        out_type=jax.ShapeDtypeStruct((NTOK, 2 * D), jnp.float32),
        scratch_types=[
            pltpu.VMEM((_W, 2 * D), jnp.float32),
            pltpu.VMEM((_W,), jnp.int32),
        ],
    )
    def k(so_hbm, gidx_hbm, oext_hbm, rows_v, idx_v):
        wid = lax.axis_index("s") * 2 + lax.axis_index("c")
        per_w = NTOK // _NW

        @pl.loop(0, per_w // _W)
        def _(ci):
            g0 = wid * per_w + ci * _W
            pltpu.sync_copy(gidx_hbm.at[pl.ds(g0, _W)], idx_v)
            pltpu.sync_copy(so_hbm.at[idx_v], rows_v)
            pltpu.sync_copy(rows_v, oext_hbm.at[pl.ds(g0, _W)])

    return k(so_ext, gidx)


def _attn_kernel(sqkv_ref, halo_ref, bmask_ref, out_ref,
                 q_ref, k_ref, v_ref, d_ref, p_ref):
    def nrm(x):
        n = jnp.sqrt(jnp.sum(x * x, axis=1, keepdims=True))
        return x / jnp.maximum(n, 1e-12)

    x2 = sqkv_ref[0].reshape(CPG * CS, 2 * D)
    halo = halo_ref[0, 0]
    q_all = x2[:, :D]
    q_ref[...] = q_all.astype(jnp.bfloat16)
    k_ref[0:CS, :] = nrm(halo[:, :D]).astype(jnp.bfloat16)
    k_ref[CS:, :] = nrm(q_all).astype(jnp.bfloat16)
    v_ref[0:CS, :] = halo[:, D:].astype(jnp.bfloat16)
    v_ref[CS:, :] = x2[:, D:].astype(jnp.bfloat16)

    for c in range(CPG):
        d_ref[c * CS:(c + 1) * CS, :] = lax.dot_general(
            q_ref[c * CS:(c + 1) * CS, :], k_ref[c * CS:(c + 2) * CS, :],
            (((1,), (1,)), ((), ())), preferred_element_type=jnp.float32)

    dots = d_ref[...] * SCALE
    ii = lax.broadcasted_iota(jnp.int32, (CPG * CS, 2 * CS), 0)
    jj = lax.broadcasted_iota(jnp.int32, (CPG * CS, 2 * CS), 1)
    eye = jj == (ii % CS) + CS
    dots = jnp.where(eye, -50000.0, dots)
    bpad = jnp.pad(bmask_ref[0, 0], ((0, CPG * CS - CS), (0, CS)))
    dots = jnp.where(bpad > 0.5, -50000.0, dots)
    m = jnp.max(dots, axis=1, keepdims=True)
    pexp = jnp.exp(dots - m)
    ssum = jnp.sum(pexp, axis=1, keepdims=True)
    lse = m + jnp.log(ssum)
    p_ref[...] = (pexp / ssum).astype(jnp.bfloat16)
    out_ref[0, :, :, D:] = jnp.broadcast_to(
        lse.reshape(CPG, CS, 1), (CPG, CS, D))

    for c in range(CPG):
        out_ref[0, c, :, :D] = lax.dot_general(
            p_ref[c * CS:(c + 1) * CS, :], v_ref[c * CS:(c + 2) * CS, :],
            (((1,), (0,)), ((), ())), preferred_element_type=jnp.float32)


def _attention(sqkv4, bmask):
    return pl.pallas_call(
        _attn_kernel,
        grid=(B, NGRP),
        in_specs=[
            pl.BlockSpec((1, CPG, CS, 2 * D), lambda b, g: (b, g, 0, 0)),
            pl.BlockSpec((1, 1, CS, 2 * D),
                         lambda b, g: (b, (g * CPG - 1) % NC, 0, 0)),
            pl.BlockSpec((1, 1, CS, CS), lambda b, g: (b, g, 0, 0)),
        ],
        out_specs=pl.BlockSpec((1, CPG, CS, 2 * D), lambda b, g: (b, g, 0, 0)),
        out_shape=jax.ShapeDtypeStruct((B, NC, CS, 2 * D), jnp.float32),
        scratch_shapes=[
            pltpu.VMEM((CPG * CS, D), jnp.bfloat16),
            pltpu.VMEM(((CPG + 1) * CS, D), jnp.bfloat16),
            pltpu.VMEM(((CPG + 1) * CS, D), jnp.bfloat16),
            pltpu.VMEM((CPG * CS, 2 * CS), jnp.float32),
            pltpu.VMEM((CPG * CS, 2 * CS), jnp.bfloat16),
        ],
    )(sqkv4, sqkv4, bmask)


_T = 512


def _combine_kernel(oext_ref, out_ref):
    x = oext_ref[0]
    o = x[:, :, :D]
    l = x[:, :, D:D + 1]
    m = jnp.max(l, axis=0, keepdims=True)
    w = jnp.exp(l - m)
    s = jnp.sum(w, axis=0)
    acc = jnp.sum(o * w, axis=0)
    out_ref[0] = acc / s


def _combine(o_ext4):
    return pl.pallas_call(
        _combine_kernel,
        grid=(B, S // _T),
        in_specs=[pl.BlockSpec((1, H, _T, 2 * D), lambda b, t: (b, 0, t, 0))],
        out_specs=pl.BlockSpec((1, _T, D), lambda b, t: (b, t, 0)),
        out_shape=jax.ShapeDtypeStruct((B, S, D), jnp.float32),
    )(o_ext4)


def kernel(qk, v, rotations):
    rot2 = rotations.reshape(D, H * 32)
    gdest4, bmask, qkv3 = _hash_dest(qk, v, rot2)
    gdest = gdest4.reshape(NTOK)

    sqkv = _sc_scatter(qkv3.reshape(B * S, 2 * D), gdest)

    sqkv4 = sqkv.reshape(B, NC, CS, 2 * D)
    so_ext = _attention(sqkv4, bmask)

    o_ext = _sc_gather(so_ext.reshape(NTOK, 2 * D), gdest)
    out = _combine(o_ext.reshape(B, H, S, 2 * D))
    return out


# speedup vs baseline: 11.9393x; 1.0861x over previous
import functools

import jax
import jax.numpy as jnp
from jax import lax
from jax.experimental import pallas as pl
from jax.experimental.pallas import tpu as pltpu
from jax.experimental.pallas import tpu_sc as plsc

B = 8
S = 4096
D = 64
H = 8
NBK = 64
CS = 64
NC = H * S // CS
CPG = 64
NGRP = NC // CPG
NTOK = B * H * S
SCALE = D ** -0.5

def _hash_dest_kernel(qk_ref, v_ref, rot_ref, gdest_ref, bmask_ref, qkv_ref):
    b = pl.program_id(0)
    x = qk_ref[0]
    rot = rot_ref[...]
    qkv_ref[0] = jnp.concatenate([x, v_ref[0]], axis=1)
    rotated = jnp.dot(x, rot, preferred_element_type=jnp.float32)

    SB = 8
    NSB = S // SB
    io_r = lax.broadcasted_iota(jnp.int32, (NBK, NBK), 0)
    io_c = lax.broadcasted_iota(jnp.int32, (NBK, NBK), 1)
    upper = (io_r < io_c).astype(jnp.float32)
    io_R = lax.broadcasted_iota(jnp.int32, (NSB, NSB), 0)
    io_C = lax.broadcasted_iota(jnp.int32, (NSB, NSB), 1)
    lowerS = (io_C < io_R).astype(jnp.float32)
    iota_v = lax.broadcasted_iota(jnp.int32, (NSB, SB, NBK), 2)
    io_j = lax.broadcasted_iota(jnp.int32, (NSB, SB, SB), 1)
    io_k = lax.broadcasted_iota(jnp.int32, (NSB, SB, SB), 2)
    jlt = io_k < io_j

    dests = []
    for h in range(H):
        rh = rotated[:, h * 32:(h + 1) * 32]
        full = jnp.concatenate([rh, -rh], axis=1)
        full3 = full.reshape(NSB, SB, NBK)
        mx = jnp.max(full3, axis=2, keepdims=True)
        bucket3 = jnp.min(jnp.where(full3 == mx, iota_v, NBK),
                          axis=2, keepdims=True)

        oh3 = (bucket3 == iota_v).astype(jnp.float32)
        cnt = jnp.sum(oh3, axis=1)
        hist = jnp.sum(cnt, axis=0, keepdims=True)
        start = jnp.dot(hist, upper, preferred_element_type=jnp.float32)
        cnt_before = jnp.dot(lowerS, cnt, preferred_element_type=jnp.float32)

        b_row = jnp.swapaxes(bucket3, 1, 2)
        cmp = jnp.logical_and(bucket3 == b_row, jlt)
        rank = jnp.sum(cmp.astype(jnp.float32), axis=2, keepdims=True)

        base = start + cnt_before
        sel = jnp.sum(base[:, None, :] * oh3, axis=2, keepdims=True)
        dest = (sel + rank).astype(jnp.int32)
        dests.append(dest)
        gdest_ref[0, h] = dest[:, :, 0] + (b * H + h) * S

    for h in range(H):
        dcur = dests[h]
        dprev = dests[(h - 1) % H]
        a = (dcur == iota_v).astype(jnp.float32).reshape(S, NBK)
        bb = (dprev == iota_v + (S - CS)).astype(jnp.float32).reshape(S, NBK)
        m = lax.dot_general(a, bb, (((0,), (0,)), ((), ())),
                            preferred_element_type=jnp.float32)
        bmask_ref[0, h] = m


def _hash_dest(qk, v, rot2):
    nb = qk.shape[0]
    return pl.pallas_call(
        _hash_dest_kernel,
        grid=(nb,),
        in_specs=[
            pl.BlockSpec((1, S, D), lambda b: (b, 0, 0)),
            pl.BlockSpec((1, S, D), lambda b: (b, 0, 0)),
            pl.BlockSpec((D, H * 32), lambda b: (0, 0)),
        ],
        out_specs=[
            pl.BlockSpec((1, H, S // 8, 8), lambda b: (b, 0, 0, 0)),
            pl.BlockSpec((1, H, CS, CS), lambda b: (b, 0, 0, 0)),
            pl.BlockSpec((1, S, 2 * D), lambda b: (b, 0, 0)),
        ],
        out_shape=[
            jax.ShapeDtypeStruct((nb, H, S // 8, 8), jnp.int32),
            jax.ShapeDtypeStruct((nb, H, CS, CS), jnp.float32),
            jax.ShapeDtypeStruct((nb, S, 2 * D), jnp.float32),
        ],
    )(qk, v, rot2)


_NW = 32
_W = 256


def _sc_mesh():
    return plsc.VectorSubcoreMesh(core_axis_name="c", subcore_axis_name="s")


def _sc_scatter(qkv, gidx):
    nb = qkv.shape[0] // S
    tok_pw = nb * S // _NW
    segs = S // tok_pw

    @functools.partial(
        pl.kernel,
        mesh=_sc_mesh(),
        out_type=jax.ShapeDtypeStruct((nb * H * S, 2 * D), jnp.float32),
        scratch_types=[
            pltpu.VMEM((_W, 2 * D), jnp.float32),
            pltpu.VMEM((_W,), jnp.int32),
        ],
    )
    def k(qkv_hbm, gidx_hbm, sqkv_hbm, rows_v, idx_v):
        wid = lax.axis_index("s") * 2 + lax.axis_index("c")
        b = wid // segs
        seg = wid % segs

        @pl.loop(0, tok_pw // _W)
        def _(ci):
            t0 = seg * tok_pw + ci * _W
            pltpu.sync_copy(qkv_hbm.at[pl.ds(b * S + t0, _W)], rows_v)
            for h in range(H):
                pltpu.sync_copy(gidx_hbm.at[pl.ds((b * H + h) * S + t0, _W)],
                                idx_v)
                pltpu.sync_copy(rows_v, sqkv_hbm.at[idx_v])

    return k(qkv, gidx)


def _sc_gather(so_ext, gidx):
    ntok = so_ext.shape[0]

    @functools.partial(
        pl.kernel,
        mesh=_sc_mesh(),
        out_type=jax.ShapeDtypeStruct((ntok, 2 * D), jnp.float32),
        scratch_types=[
            pltpu.VMEM((_W, 2 * D), jnp.float32),
            pltpu.VMEM((_W,), jnp.int32),
        ],
    )
    def k(so_hbm, gidx_hbm, oext_hbm, rows_v, idx_v):
        wid = lax.axis_index("s") * 2 + lax.axis_index("c")
        per_w = ntok // _NW

        @pl.loop(0, per_w // _W)
        def _(ci):
            g0 = wid * per_w + ci * _W
            pltpu.sync_copy(gidx_hbm.at[pl.ds(g0, _W)], idx_v)
            pltpu.sync_copy(so_hbm.at[idx_v], rows_v)
            pltpu.sync_copy(rows_v, oext_hbm.at[pl.ds(g0, _W)])

    return k(so_ext, gidx)


def _attn_kernel(sqkv_ref, halo_ref, bmask_ref, out_ref,
                 q_ref, k_ref, v_ref, d_ref, p_ref):
    def nrm(x):
        n = jnp.sqrt(jnp.sum(x * x, axis=1, keepdims=True))
        return x / jnp.maximum(n, 1e-12)

    x2 = sqkv_ref[0].reshape(CPG * CS, 2 * D)
    halo = halo_ref[0, 0]
    q_all = x2[:, :D]
    q_ref[...] = q_all.astype(jnp.bfloat16)
    k_ref[0:CS, :] = nrm(halo[:, :D]).astype(jnp.bfloat16)
    k_ref[CS:, :] = nrm(q_all).astype(jnp.bfloat16)
    v_ref[0:CS, :] = halo[:, D:].astype(jnp.bfloat16)
    v_ref[CS:, :] = x2[:, D:].astype(jnp.bfloat16)

    for c in range(CPG):
        d_ref[c * CS:(c + 1) * CS, :] = lax.dot_general(
            q_ref[c * CS:(c + 1) * CS, :], k_ref[c * CS:(c + 2) * CS, :],
            (((1,), (1,)), ((), ())), preferred_element_type=jnp.float32)

    dots = d_ref[...] * SCALE
    ii = lax.broadcasted_iota(jnp.int32, (CPG * CS, 2 * CS), 0)
    jj = lax.broadcasted_iota(jnp.int32, (CPG * CS, 2 * CS), 1)
    eye = jj == (ii % CS) + CS
    dots = jnp.where(eye, -50000.0, dots)
    bpad = jnp.pad(bmask_ref[0, 0], ((0, CPG * CS - CS), (0, CS)))
    dots = jnp.where(bpad > 0.5, -50000.0, dots)
    m = jnp.max(dots, axis=1, keepdims=True)
    pexp = jnp.exp(dots - m)
    ssum = jnp.sum(pexp, axis=1, keepdims=True)
    lse = m + jnp.log(ssum)
    p_ref[...] = (pexp / ssum).astype(jnp.bfloat16)
    out_ref[0, :, :, D:] = jnp.broadcast_to(
        lse.reshape(CPG, CS, 1), (CPG, CS, D))

    for c in range(CPG):
        out_ref[0, c, :, :D] = lax.dot_general(
            p_ref[c * CS:(c + 1) * CS, :], v_ref[c * CS:(c + 2) * CS, :],
            (((1,), (0,)), ((), ())), preferred_element_type=jnp.float32)


def _attention(sqkv4, bmask):
    nb = sqkv4.shape[0]
    return pl.pallas_call(
        _attn_kernel,
        grid=(nb, NGRP),
        in_specs=[
            pl.BlockSpec((1, CPG, CS, 2 * D), lambda b, g: (b, g, 0, 0)),
            pl.BlockSpec((1, 1, CS, 2 * D),
                         lambda b, g: (b, (g * CPG - 1) % NC, 0, 0)),
            pl.BlockSpec((1, 1, CS, CS), lambda b, g: (b, g, 0, 0)),
        ],
        out_specs=pl.BlockSpec((1, CPG, CS, 2 * D), lambda b, g: (b, g, 0, 0)),
        out_shape=jax.ShapeDtypeStruct((nb, NC, CS, 2 * D), jnp.float32),
        scratch_shapes=[
            pltpu.VMEM((CPG * CS, D), jnp.bfloat16),
            pltpu.VMEM(((CPG + 1) * CS, D), jnp.bfloat16),
            pltpu.VMEM(((CPG + 1) * CS, D), jnp.bfloat16),
            pltpu.VMEM((CPG * CS, 2 * CS), jnp.float32),
            pltpu.VMEM((CPG * CS, 2 * CS), jnp.bfloat16),
        ],
    )(sqkv4, sqkv4, bmask)


_T = 512


def _combine_kernel(oext_ref, out_ref):
    x = oext_ref[0]
    o = x[:, :, :D]
    l = x[:, :, D:D + 1]
    m = jnp.max(l, axis=0, keepdims=True)
    w = jnp.exp(l - m)
    s = jnp.sum(w, axis=0)
    acc = jnp.sum(o * w, axis=0)
    out_ref[0] = acc / s


def _combine(o_ext4):
    nb = o_ext4.shape[0]
    return pl.pallas_call(
        _combine_kernel,
        grid=(nb, S // _T),
        in_specs=[pl.BlockSpec((1, H, _T, 2 * D), lambda b, t: (b, 0, t, 0))],
        out_specs=pl.BlockSpec((1, _T, D), lambda b, t: (b, t, 0)),
        out_shape=jax.ShapeDtypeStruct((nb, S, D), jnp.float32),
    )(o_ext4)


def kernel(qk, v, rotations):
    rot2 = rotations.reshape(D, H * 32)
    hb = B // 2
    gdA, bmA, qvA = _hash_dest(qk[:hb], v[:hb], rot2)
    gdB, bmB, qvB = _hash_dest(qk[hb:], v[hb:], rot2)
    gdA = gdA.reshape(hb * H * S)
    gdB = gdB.reshape(hb * H * S)
    sA = _sc_scatter(qvA.reshape(hb * S, 2 * D), gdA)
    soA = _attention(sA.reshape(hb, NC, CS, 2 * D), bmA)
    sB = _sc_scatter(qvB.reshape(hb * S, 2 * D), gdB)
    oA = _sc_gather(soA.reshape(hb * H * S, 2 * D), gdA)
    soB = _attention(sB.reshape(hb, NC, CS, 2 * D), bmB)
    outA = _combine(oA.reshape(hb, H, S, 2 * D))
    oB = _sc_gather(soB.reshape(hb * H * S, 2 * D), gdB)
    outB = _combine(oB.reshape(hb, H, S, 2 * D))
    return jnp.concatenate([outA, outB], axis=0)
